# Initial kernel scaffold; baseline (speedup 1.0000x reference)
import jax
import jax.numpy as jnp
from jax.experimental import pallas as pl


def kernel(embeddings, item_ids, queue_embeddings, queue_item_ids, ptr):
    raise NotImplementedError("write your pallas kernel here")


# speedup vs baseline: 1.5887x; 1.5887x over previous
import jax
import jax.numpy as jnp
from jax import lax
from jax.experimental import pallas as pl
from jax.experimental.pallas import tpu as pltpu

_R = 2048


def _body(p_ref, er, ir, qe, qi, se, ne, si, ni):
    cap = pl.num_programs(0) * _R
    p = p_ref[0]
    b = p_ref[1]
    g = pl.program_id(0)
    t0 = jnp.mod(g * _R - p, cap)

    rows = lax.broadcasted_iota(jnp.int32, (_R, qe.shape[1]), 0)
    tt = t0 + rows
    tt = jnp.where(tt >= cap, tt - cap, tt)
    mask = tt < b

    rows2 = lax.broadcasted_iota(jnp.int32, (_R, 2), 0)
    tt2 = t0 + rows2
    tt2 = jnp.where(tt2 >= cap, tt2 - cap, tt2)
    mask2 = tt2 < b

    se[...] = qe[...]
    si[...] = qi[...]
    ne[...] = jnp.where(mask, er[...], qe[...])
    ni[...] = jnp.where(mask2, ir[...], qi[...])


def kernel(embeddings, item_ids, queue_embeddings, queue_item_ids, ptr):
    C, D = queue_embeddings.shape
    B = embeddings.shape[0]
    G = C // _R
    WB = B // _R

    p = jnp.asarray(jnp.mod(ptr, C), jnp.int32)
    a = jnp.mod(p, _R)

    emb2 = jnp.concatenate([embeddings, embeddings], axis=0)
    emb_roll = lax.dynamic_slice(emb2, (B - a, jnp.int32(0)), (B, D))
    ids32 = lax.bitcast_convert_type(item_ids, jnp.int32)
    qids32 = lax.bitcast_convert_type(queue_item_ids, jnp.int32)
    ids2 = jnp.concatenate([ids32, ids32], axis=0)
    ids_roll = lax.dynamic_slice(ids2, (B - a, jnp.int32(0)), (B, 2))

    scal = jnp.stack([p, jnp.int32(B)])

    def win_map(g, pr):
        jm = jnp.mod(g - pr[0] // _R, G)
        return (jnp.mod(jnp.minimum(jm, WB), WB), jnp.int32(0))

    def id_map(g, pr):
        return (g, jnp.int32(0))

    grid_spec = pltpu.PrefetchScalarGridSpec(
        num_scalar_prefetch=1,
        grid=(G,),
        in_specs=[
            pl.BlockSpec((_R, D), win_map),
            pl.BlockSpec((_R, 2), win_map),
            pl.BlockSpec((_R, D), id_map),
            pl.BlockSpec((_R, 2), id_map),
        ],
        out_specs=[
            pl.BlockSpec((_R, D), id_map),
            pl.BlockSpec((_R, D), id_map),
            pl.BlockSpec((_R, 2), id_map),
            pl.BlockSpec((_R, 2), id_map),
        ],
    )

    se, ne, si, ni = pl.pallas_call(
        _body,
        grid_spec=grid_spec,
        out_shape=[
            jax.ShapeDtypeStruct((C, D), jnp.float32),
            jax.ShapeDtypeStruct((C, D), jnp.float32),
            jax.ShapeDtypeStruct((C, 2), jnp.int32),
            jax.ShapeDtypeStruct((C, 2), jnp.int32),
        ],
    )(scal, emb_roll, ids_roll, queue_embeddings, qids32)

    sampled_ids = lax.bitcast_convert_type(si, jnp.int64)
    new_ids = lax.bitcast_convert_type(ni, jnp.int64)
    return (se, sampled_ids, ne, new_ids)
